# Initial kernel scaffold; baseline (speedup 1.0000x reference)
import jax
import jax.numpy as jnp
from jax.experimental import pallas as pl


def kernel(x, edge_index, batch, W_emb, b_emb, W_g1, b_g1, aW1, ab1, aW2, ab2, vW1, vb1, vW2, vb2, mW1, mb1, mW2, mb2, edge_weights):
    raise NotImplementedError("write your pallas kernel here")


# speedup vs baseline: 19.8890x; 19.8890x over previous
import functools

import jax
import jax.numpy as jnp
from jax import lax
from jax.experimental import pallas as pl
from jax.experimental.pallas import tpu as pltpu
from jax.experimental.pallas import tpu_sc as plsc

G = 16
NPG = 1024
N = G * NPG
E = 262144
DIN = 256
H = 256
DOUT = 64
V = 128

NC = 2
NS = 16
NW = NC * NS

NCHUNK = 4
CW = H // NCHUNK


_DEG_EPT = E // NW
_DEG_NB = _DEG_EPT // 128

_sc_mesh = plsc.VectorSubcoreMesh(core_axis_name="c", subcore_axis_name="s")


@functools.partial(
    pl.kernel,
    out_type=jax.ShapeDtypeStruct((NC, N, 16), jnp.float32),
    mesh=_sc_mesh,
    compiler_params=pltpu.CompilerParams(use_tc_tiling_on_sc=False),
    scratch_types=[
        pltpu.VMEM_SHARED((N, 16), jnp.float32),
        pltpu.VMEM((_DEG_NB, 128), jnp.int32),
        pltpu.VMEM((128, 16), jnp.float32),
    ],
)
def _sc_degree(dst_hbm, ones_hbm, zeros_hbm, deg_hbm, acc, didx, obuf):
    c = lax.axis_index("c")
    s = lax.axis_index("s")
    w = s * NC + c
    pltpu.sync_copy(zeros_hbm, acc.at[pl.ds(s * NPG, NPG)])
    pltpu.sync_copy(ones_hbm, obuf)
    pltpu.sync_copy(dst_hbm.at[w], didx)
    plsc.subcore_barrier()

    def body(j, carry):
        pltpu.sync_copy(obuf, acc.at[didx.at[j]], add=True)
        return carry

    lax.fori_loop(0, _DEG_NB, body, 0)
    plsc.subcore_barrier()
    pltpu.sync_copy(acc.at[pl.ds(s * NPG, NPG)],
                    deg_hbm.at[c, pl.ds(s * NPG, NPG)])


def _run_sc_degree(dst):
    dst_r = dst.reshape(NW, _DEG_NB, 128)
    ones = jnp.ones((128, 16), jnp.float32)
    zeros = jnp.zeros((NPG, 16), jnp.float32)
    return _sc_degree(dst_r, ones, zeros)


_MSG_EPT = E // NS
_MSG_NB = _MSG_EPT // 128


@functools.partial(
    pl.kernel,
    out_type=jax.ShapeDtypeStruct((NCHUNK * N, CW), jnp.float32),
    mesh=_sc_mesh,
    compiler_params=pltpu.CompilerParams(use_tc_tiling_on_sc=False),
    scratch_types=[
        pltpu.VMEM_SHARED((N, CW), jnp.float32),
        pltpu.VMEM((_MSG_EPT,), jnp.int32),
        pltpu.VMEM((_MSG_EPT,), jnp.int32),
        pltpu.VMEM((_MSG_NB, 128), jnp.int32),
        pltpu.VMEM((128, CW), jnp.float32),
        pltpu.VMEM((128, CW), jnp.float32),
        pltpu.SemaphoreType.DMA,
        pltpu.SemaphoreType.DMA,
    ],
)
def _sc_scatter(src_hbm, dst_hbm, hws_hbm, out_hbm, acc,
                sidx, sadj, didx, rb0, rb1, sem0, sem1):
    c = lax.axis_index("c")
    s = lax.axis_index("s")
    pltpu.sync_copy(src_hbm.at[s], sidx)
    pltpu.sync_copy(dst_hbm.at[s], didx)

    def gat(b, rb, sem):
        return pltpu.async_copy(hws_hbm.at[sadj.at[pl.ds(b * 128, 128)]],
                                rb, sem)

    for j in range(2):
        off = (2 * c + j) * N

        def adj(i, carry):
            sl = pl.ds(i * 16, 16)
            sadj[sl] = sidx[sl] + off
            return carry

        lax.fori_loop(0, _MSG_EPT // 16, adj, 0)

        pltpu.sync_copy(hws_hbm.at[pl.ds(off + s * NPG, NPG)],
                        acc.at[pl.ds(s * NPG, NPG)])
        plsc.subcore_barrier()

        gat(0, rb0, sem0)

        def wat(b, rb, sem):
            pltpu.make_async_copy(hws_hbm.at[sadj.at[pl.ds(b * 128, 128)]],
                                  rb, sem).wait()

        def body(i2, carry):
            b0 = 2 * i2
            gat(b0 + 1, rb1, sem1)
            wat(b0, rb0, sem0)
            pltpu.sync_copy(rb0, acc.at[didx.at[b0]], add=True)

            @pl.when(i2 < _MSG_NB // 2 - 1)
            def _():
                gat(b0 + 2, rb0, sem0)

            wat(b0 + 1, rb1, sem1)
            pltpu.sync_copy(rb1, acc.at[didx.at[b0 + 1]], add=True)
            return carry

        lax.fori_loop(0, _MSG_NB // 2, body, 0)
        plsc.subcore_barrier()
        pltpu.sync_copy(acc.at[pl.ds(s * NPG, NPG)],
                        out_hbm.at[pl.ds(off + s * NPG, NPG)])
        plsc.subcore_barrier()


def _run_sc_scatter(src, dst, hws4):
    src_r = src.reshape(NS, _MSG_EPT)
    dst_r = dst.reshape(NS, _MSG_NB, 128)
    return _sc_scatter(src_r, dst_r, hws4)


_K1_BLK = 256
_K1_GRID = N // _K1_BLK


def _tc_k1_body(x_ref, wemb_ref, bemb_ref, wg1_ref, dega_ref, degb_ref,
                out_ref):
    xb = x_ref[...]
    hb = jnp.dot(xb, wemb_ref[...], preferred_element_type=jnp.float32)
    hb = hb + bemb_ref[...]
    hw = jnp.dot(hb, wg1_ref[...], preferred_element_type=jnp.float32)
    deg = dega_ref[0, :, 0:1] + degb_ref[0, :, 0:1] + 1.0
    hws = hw * lax.rsqrt(deg)
    for cidx in range(NCHUNK):
        out_ref[cidx] = hws[:, cidx * CW:(cidx + 1) * CW]


def _run_tc_k1(x, W_emb, b_emb, W_g1, deg2):
    da = deg2[0].reshape(_K1_GRID, _K1_BLK, 16)
    db = deg2[1].reshape(_K1_GRID, _K1_BLK, 16)
    full = lambda shape: pl.BlockSpec(shape, lambda i: (0,) * len(shape))
    out = pl.pallas_call(
        _tc_k1_body,
        grid=(_K1_GRID,),
        in_specs=[
            pl.BlockSpec((_K1_BLK, DIN), lambda i: (i, 0)),
            full((DIN, H)),
            full((1, H)),
            full((H, H)),
            pl.BlockSpec((1, _K1_BLK, 16), lambda i: (i, 0, 0)),
            pl.BlockSpec((1, _K1_BLK, 16), lambda i: (i, 0, 0)),
        ],
        out_specs=pl.BlockSpec((NCHUNK, _K1_BLK, CW), lambda i: (0, i, 0)),
        out_shape=jax.ShapeDtypeStruct((NCHUNK, N, CW), jnp.float32),
    )(x, W_emb, b_emb.reshape(1, H), W_g1, da, db)
    return out.reshape(NCHUNK * N, CW)


def _tc_k2_body(op_ref, dega_ref, degb_ref, bg1_ref, aw1_ref, ab1_ref,
                aw2_ref, ab2_ref, virt_ref, ewt_ref, vw1_ref, vb1_ref,
                vw2_ref, vb2_ref, mw1_ref, mb1_ref, mw2_ref, mb2_ref,
                out_ref):
    parts = [op_ref[cidx, 0] for cidx in range(NCHUNK)]
    pre = jnp.concatenate(parts, axis=1)
    deg = dega_ref[0, :, 0:1] + degb_ref[0, :, 0:1] + 1.0
    gx = jax.nn.relu(pre * lax.rsqrt(deg) + bg1_ref[...])
    af = jnp.dot(gx, aw1_ref[...], preferred_element_type=jnp.float32)
    af = jax.nn.relu(af + ab1_ref[...])
    af = jnp.dot(af, aw2_ref[...], preferred_element_type=jnp.float32)
    af = af + ab2_ref[...]
    scores = lax.dot_general(af, virt_ref[0],
                             (((1,), (1,)), ((), ())),
                             preferred_element_type=jnp.float32) * (1.0 / 16.0)
    ew = ewt_ref[0] * (1.0 + jax.nn.sigmoid(scores))
    rs = jnp.sum(ew, axis=1, keepdims=True)
    rs = jnp.where(rs == 0.0, 1.0, rs)
    ew = ew / rs
    vn = lax.dot_general(ew, gx, (((0,), (0,)), ((), ())),
                         preferred_element_type=jnp.float32)
    h1 = jnp.dot(vn, vw1_ref[...], preferred_element_type=jnp.float32)
    h1 = jax.nn.relu(h1 + vb1_ref[...])
    h1 = jnp.dot(h1, vw2_ref[...], preferred_element_type=jnp.float32)
    h1 = h1 + vb2_ref[...]
    gf = jnp.mean(h1, axis=0, keepdims=True)
    o = jnp.dot(gf, mw1_ref[...], preferred_element_type=jnp.float32)
    o = jax.nn.relu(o + mb1_ref[...])
    o = jnp.dot(o, mw2_ref[...], preferred_element_type=jnp.float32)
    out_ref[0] = o + mb2_ref[...]


def _run_tc_k2(outpre4, deg2, b_g1, aW1, ab1, aW2, ab2, virt, edge_weights,
               vW1, vb1, vW2, vb2, mW1, mb1, mW2, mb2):
    op = outpre4.reshape(NCHUNK, G, NPG, CW)
    da = deg2[0].reshape(G, NPG, 16)
    db = deg2[1].reshape(G, NPG, 16)
    full = lambda shape: pl.BlockSpec(shape, lambda g: (0,) * len(shape))
    out = pl.pallas_call(
        _tc_k2_body,
        grid=(G,),
        in_specs=[
            pl.BlockSpec((NCHUNK, 1, NPG, CW), lambda g: (0, g, 0, 0)),
            pl.BlockSpec((1, NPG, 16), lambda g: (g, 0, 0)),
            pl.BlockSpec((1, NPG, 16), lambda g: (g, 0, 0)),
            full((1, H)),
            full((H, H)),
            full((1, H)),
            full((H, H)),
            full((1, H)),
            pl.BlockSpec((1, V, H), lambda g: (g, 0, 0)),
            pl.BlockSpec((1, NPG, V), lambda g: (g, 0, 0)),
            full((H, H)),
            full((1, H)),
            full((H, H)),
            full((1, H)),
            full((H, H)),
            full((1, H)),
            full((H, DOUT)),
            full((1, DOUT)),
        ],
        out_specs=pl.BlockSpec((1, 1, DOUT), lambda g: (g, 0, 0)),
        out_shape=jax.ShapeDtypeStruct((G, 1, DOUT), jnp.float32),
    )(op, da, db, b_g1.reshape(1, H), aW1, ab1.reshape(1, H), aW2,
      ab2.reshape(1, H), virt, edge_weights, vW1, vb1.reshape(1, H), vW2,
      vb2.reshape(1, H), mW1, mb1.reshape(1, H), mW2, mb2.reshape(1, DOUT))
    return out.reshape(G, DOUT)


def kernel(x, edge_index, batch, W_emb, b_emb, W_g1, b_g1, aW1, ab1, aW2,
           ab2, vW1, vb1, vW2, vb2, mW1, mb1, mW2, mb2, edge_weights):
    src = edge_index[0]
    dst = edge_index[1]
    deg2 = _run_sc_degree(dst)
    hws4 = _run_tc_k1(x, W_emb, b_emb, W_g1, deg2)
    outpre4 = _run_sc_scatter(src, dst, hws4)
    virt = jax.random.normal(jax.random.key(42), (G, V, H), dtype=jnp.float32)
    virt = virt / jnp.linalg.norm(virt, axis=2, keepdims=True)
    return _run_tc_k2(outpre4, deg2, b_g1, aW1, ab1, aW2, ab2, virt,
                      edge_weights, vW1, vb1, vW2, vb2, mW1, mb1, mW2, mb2)


# speedup vs baseline: 20.7388x; 1.0427x over previous
import functools

import jax
import jax.numpy as jnp
from jax import lax
from jax.experimental import pallas as pl
from jax.experimental.pallas import tpu as pltpu
from jax.experimental.pallas import tpu_sc as plsc

G = 16
NPG = 1024
N = G * NPG
E = 262144
DIN = 256
H = 256
DOUT = 64
V = 128

NC = 2
NS = 16
NW = NC * NS

NCHUNK = 4
CW = H // NCHUNK


_DEG_EPT = E // NW
_DEG_NB = _DEG_EPT // 128

_sc_mesh = plsc.VectorSubcoreMesh(core_axis_name="c", subcore_axis_name="s")


@functools.partial(
    pl.kernel,
    out_type=jax.ShapeDtypeStruct((NC, N, 16), jnp.float32),
    mesh=_sc_mesh,
    compiler_params=pltpu.CompilerParams(use_tc_tiling_on_sc=False),
    scratch_types=[
        pltpu.VMEM_SHARED((N, 16), jnp.float32),
        pltpu.VMEM((_DEG_NB, 128), jnp.int32),
        pltpu.VMEM((128, 16), jnp.float32),
    ],
)
def _sc_degree(dst_hbm, ones_hbm, zeros_hbm, deg_hbm, acc, didx, obuf):
    c = lax.axis_index("c")
    s = lax.axis_index("s")
    w = s * NC + c
    pltpu.sync_copy(zeros_hbm, acc.at[pl.ds(s * NPG, NPG)])
    pltpu.sync_copy(ones_hbm, obuf)
    pltpu.sync_copy(dst_hbm.at[w], didx)
    plsc.subcore_barrier()

    def body(j, carry):
        pltpu.sync_copy(obuf, acc.at[didx.at[j]], add=True)
        return carry

    lax.fori_loop(0, _DEG_NB, body, 0)
    plsc.subcore_barrier()
    pltpu.sync_copy(acc.at[pl.ds(s * NPG, NPG)],
                    deg_hbm.at[c, pl.ds(s * NPG, NPG)])


def _run_sc_degree(dst):
    dst_r = dst.reshape(NW, _DEG_NB, 128)
    ones = jnp.ones((128, 16), jnp.float32)
    zeros = jnp.zeros((NPG, 16), jnp.float32)
    return _sc_degree(dst_r, ones, zeros)


_MSG_EPT = E // NS
_MSG_NB = _MSG_EPT // 128


@functools.partial(
    pl.kernel,
    out_type=jax.ShapeDtypeStruct((NCHUNK * N, CW), jnp.bfloat16),
    mesh=_sc_mesh,
    compiler_params=pltpu.CompilerParams(use_tc_tiling_on_sc=False),
    scratch_types=[
        pltpu.VMEM_SHARED((N, CW), jnp.bfloat16),
        pltpu.VMEM((_MSG_EPT,), jnp.int32),
        pltpu.VMEM((_MSG_EPT,), jnp.int32),
        pltpu.VMEM((_MSG_NB, 128), jnp.int32),
        pltpu.VMEM((128, CW), jnp.bfloat16),
        pltpu.VMEM((128, CW), jnp.bfloat16),
        pltpu.SemaphoreType.DMA,
        pltpu.SemaphoreType.DMA,
    ],
)
def _sc_scatter(src_hbm, dst_hbm, hws_hbm, out_hbm, acc,
                sidx, sadj, didx, rb0, rb1, sem0, sem1):
    c = lax.axis_index("c")
    s = lax.axis_index("s")
    pltpu.sync_copy(src_hbm.at[s], sidx)
    pltpu.sync_copy(dst_hbm.at[s], didx)

    def gat(b, rb, sem):
        return pltpu.async_copy(hws_hbm.at[sadj.at[pl.ds(b * 128, 128)]],
                                rb, sem)

    for j in range(2):
        off = (2 * c + j) * N

        def adj(i, carry):
            sl = pl.ds(i * 16, 16)
            sadj[sl] = sidx[sl] + off
            return carry

        lax.fori_loop(0, _MSG_EPT // 16, adj, 0)

        pltpu.sync_copy(hws_hbm.at[pl.ds(off + s * NPG, NPG)],
                        acc.at[pl.ds(s * NPG, NPG)])
        plsc.subcore_barrier()

        gat(0, rb0, sem0)

        def wat(b, rb, sem):
            pltpu.make_async_copy(hws_hbm.at[sadj.at[pl.ds(b * 128, 128)]],
                                  rb, sem).wait()

        def body(i2, carry):
            b0 = 2 * i2
            gat(b0 + 1, rb1, sem1)
            wat(b0, rb0, sem0)
            pltpu.sync_copy(rb0, acc.at[didx.at[b0]], add=True)

            @pl.when(i2 < _MSG_NB // 2 - 1)
            def _():
                gat(b0 + 2, rb0, sem0)

            wat(b0 + 1, rb1, sem1)
            pltpu.sync_copy(rb1, acc.at[didx.at[b0 + 1]], add=True)
            return carry

        lax.fori_loop(0, _MSG_NB // 2, body, 0)
        plsc.subcore_barrier()
        pltpu.sync_copy(acc.at[pl.ds(s * NPG, NPG)],
                        out_hbm.at[pl.ds(off + s * NPG, NPG)])
        plsc.subcore_barrier()


def _run_sc_scatter(src, dst, hws4):
    src_r = src.reshape(NS, _MSG_EPT)
    dst_r = dst.reshape(NS, _MSG_NB, 128)
    return _sc_scatter(src_r, dst_r, hws4)


_K1_BLK = 256
_K1_GRID = N // _K1_BLK


def _tc_k1_body(x_ref, wemb_ref, bemb_ref, wg1_ref, dega_ref, degb_ref,
                out_ref):
    xb = x_ref[...].astype(jnp.bfloat16)
    hb = jnp.dot(xb, wemb_ref[...].astype(jnp.bfloat16),
                 preferred_element_type=jnp.float32)
    hb = (hb + bemb_ref[...]).astype(jnp.bfloat16)
    hw = jnp.dot(hb, wg1_ref[...].astype(jnp.bfloat16),
                 preferred_element_type=jnp.float32)
    deg = dega_ref[0, :, 0:1] + degb_ref[0, :, 0:1] + 1.0
    hws = (hw * lax.rsqrt(deg)).astype(jnp.bfloat16)
    for cidx in range(NCHUNK):
        out_ref[cidx] = hws[:, cidx * CW:(cidx + 1) * CW]


def _run_tc_k1(x, W_emb, b_emb, W_g1, deg2):
    da = deg2[0].reshape(_K1_GRID, _K1_BLK, 16)
    db = deg2[1].reshape(_K1_GRID, _K1_BLK, 16)
    full = lambda shape: pl.BlockSpec(shape, lambda i: (0,) * len(shape))
    out = pl.pallas_call(
        _tc_k1_body,
        grid=(_K1_GRID,),
        in_specs=[
            pl.BlockSpec((_K1_BLK, DIN), lambda i: (i, 0)),
            full((DIN, H)),
            full((1, H)),
            full((H, H)),
            pl.BlockSpec((1, _K1_BLK, 16), lambda i: (i, 0, 0)),
            pl.BlockSpec((1, _K1_BLK, 16), lambda i: (i, 0, 0)),
        ],
        out_specs=pl.BlockSpec((NCHUNK, _K1_BLK, CW), lambda i: (0, i, 0)),
        out_shape=jax.ShapeDtypeStruct((NCHUNK, N, CW), jnp.bfloat16),
    )(x, W_emb, b_emb.reshape(1, H), W_g1, da, db)
    return out.reshape(NCHUNK * N, CW)


def _tc_k2_body(op_ref, dega_ref, degb_ref, bg1_ref, aw1_ref, ab1_ref,
                aw2_ref, ab2_ref, virt_ref, ewt_ref, vw1_ref, vb1_ref,
                vw2_ref, vb2_ref, mw1_ref, mb1_ref, mw2_ref, mb2_ref,
                out_ref):
    parts = [op_ref[cidx, 0] for cidx in range(NCHUNK)]
    pre = jnp.concatenate(parts, axis=1).astype(jnp.float32)
    deg = dega_ref[0, :, 0:1] + degb_ref[0, :, 0:1] + 1.0
    gx = jax.nn.relu(pre * lax.rsqrt(deg) + bg1_ref[...])
    gxb = gx.astype(jnp.bfloat16)
    af = jnp.dot(gxb, aw1_ref[...].astype(jnp.bfloat16),
                 preferred_element_type=jnp.float32)
    af = jax.nn.relu(af + ab1_ref[...]).astype(jnp.bfloat16)
    af = jnp.dot(af, aw2_ref[...].astype(jnp.bfloat16),
                 preferred_element_type=jnp.float32)
    af = (af + ab2_ref[...]).astype(jnp.bfloat16)
    scores = lax.dot_general(af, virt_ref[0].astype(jnp.bfloat16),
                             (((1,), (1,)), ((), ())),
                             preferred_element_type=jnp.float32) * (1.0 / 16.0)
    ew = ewt_ref[0] * (1.0 + jax.nn.sigmoid(scores))
    rs = jnp.sum(ew, axis=1, keepdims=True)
    rs = jnp.where(rs == 0.0, 1.0, rs)
    ew = (ew / rs).astype(jnp.bfloat16)
    vn = lax.dot_general(ew, gxb, (((0,), (0,)), ((), ())),
                         preferred_element_type=jnp.float32)
    h1 = jnp.dot(vn.astype(jnp.bfloat16), vw1_ref[...].astype(jnp.bfloat16),
                 preferred_element_type=jnp.float32)
    h1 = jax.nn.relu(h1 + vb1_ref[...]).astype(jnp.bfloat16)
    h1 = jnp.dot(h1, vw2_ref[...].astype(jnp.bfloat16),
                 preferred_element_type=jnp.float32)
    h1 = h1 + vb2_ref[...]
    gf = jnp.mean(h1, axis=0, keepdims=True)
    o = jnp.dot(gf, mw1_ref[...], preferred_element_type=jnp.float32)
    o = jax.nn.relu(o + mb1_ref[...])
    o = jnp.dot(o, mw2_ref[...], preferred_element_type=jnp.float32)
    out_ref[0] = o + mb2_ref[...]


def _run_tc_k2(outpre4, deg2, b_g1, aW1, ab1, aW2, ab2, virt, edge_weights,
               vW1, vb1, vW2, vb2, mW1, mb1, mW2, mb2):
    op = outpre4.reshape(NCHUNK, G, NPG, CW)
    da = deg2[0].reshape(G, NPG, 16)
    db = deg2[1].reshape(G, NPG, 16)
    full = lambda shape: pl.BlockSpec(shape, lambda g: (0,) * len(shape))
    out = pl.pallas_call(
        _tc_k2_body,
        grid=(G,),
        in_specs=[
            pl.BlockSpec((NCHUNK, 1, NPG, CW), lambda g: (0, g, 0, 0)),
            pl.BlockSpec((1, NPG, 16), lambda g: (g, 0, 0)),
            pl.BlockSpec((1, NPG, 16), lambda g: (g, 0, 0)),
            full((1, H)),
            full((H, H)),
            full((1, H)),
            full((H, H)),
            full((1, H)),
            pl.BlockSpec((1, V, H), lambda g: (g, 0, 0)),
            pl.BlockSpec((1, NPG, V), lambda g: (g, 0, 0)),
            full((H, H)),
            full((1, H)),
            full((H, H)),
            full((1, H)),
            full((H, H)),
            full((1, H)),
            full((H, DOUT)),
            full((1, DOUT)),
        ],
        out_specs=pl.BlockSpec((1, 1, DOUT), lambda g: (g, 0, 0)),
        out_shape=jax.ShapeDtypeStruct((G, 1, DOUT), jnp.float32),
    )(op, da, db, b_g1.reshape(1, H), aW1, ab1.reshape(1, H), aW2,
      ab2.reshape(1, H), virt, edge_weights, vW1, vb1.reshape(1, H), vW2,
      vb2.reshape(1, H), mW1, mb1.reshape(1, H), mW2, mb2.reshape(1, DOUT))
    return out.reshape(G, DOUT)


def kernel(x, edge_index, batch, W_emb, b_emb, W_g1, b_g1, aW1, ab1, aW2,
           ab2, vW1, vb1, vW2, vb2, mW1, mb1, mW2, mb2, edge_weights):
    src = edge_index[0]
    dst = edge_index[1]
    deg2 = _run_sc_degree(dst)
    hws4 = _run_tc_k1(x, W_emb, b_emb, W_g1, deg2)
    outpre4 = _run_sc_scatter(src, dst, hws4)
    virt = jax.random.normal(jax.random.key(42), (G, V, H), dtype=jnp.float32)
    virt = virt / jnp.linalg.norm(virt, axis=2, keepdims=True)
    return _run_tc_k2(outpre4, deg2, b_g1, aW1, ab1, aW2, ab2, virt,
                      edge_weights, vW1, vb1, vW2, vb2, mW1, mb1, mW2, mb2)


# speedup vs baseline: 25.1624x; 1.2133x over previous
import functools

import jax
import jax.numpy as jnp
from jax import lax
from jax.experimental import pallas as pl
from jax.experimental.pallas import tpu as pltpu
from jax.experimental.pallas import tpu_sc as plsc

G = 16
NPG = 1024
N = G * NPG
E = 262144
DIN = 256
H = 256
DOUT = 64
V = 128

NC = 2
NS = 16
NW = NC * NS

NCHUNK = 2
CW = H // NCHUNK
CPS = NCHUNK // NC


_DEG_EPT = E // NW
_DEG_NB = _DEG_EPT // 128

_sc_mesh = plsc.VectorSubcoreMesh(core_axis_name="c", subcore_axis_name="s")


@functools.partial(
    pl.kernel,
    out_type=jax.ShapeDtypeStruct((NC, N, 16), jnp.float32),
    mesh=_sc_mesh,
    compiler_params=pltpu.CompilerParams(use_tc_tiling_on_sc=False),
    scratch_types=[
        pltpu.VMEM_SHARED((N, 16), jnp.float32),
        pltpu.VMEM((_DEG_NB, 128), jnp.int32),
        pltpu.VMEM((128, 16), jnp.float32),
    ],
)
def _sc_degree(dst_hbm, ones_hbm, zeros_hbm, deg_hbm, acc, didx, obuf):
    c = lax.axis_index("c")
    s = lax.axis_index("s")
    w = s * NC + c
    pltpu.sync_copy(zeros_hbm, acc.at[pl.ds(s * NPG, NPG)])
    pltpu.sync_copy(ones_hbm, obuf)
    pltpu.sync_copy(dst_hbm.at[w], didx)
    plsc.subcore_barrier()

    def body(j, carry):
        pltpu.sync_copy(obuf, acc.at[didx.at[j]], add=True)
        return carry

    lax.fori_loop(0, _DEG_NB, body, 0)
    plsc.subcore_barrier()
    pltpu.sync_copy(acc.at[pl.ds(s * NPG, NPG)],
                    deg_hbm.at[c, pl.ds(s * NPG, NPG)])


def _run_sc_degree(dst):
    dst_r = dst.reshape(NW, _DEG_NB, 128)
    ones = jnp.ones((128, 16), jnp.float32)
    zeros = jnp.zeros((NPG, 16), jnp.float32)
    return _sc_degree(dst_r, ones, zeros)


_MSG_EPT = E // NS
_MSG_NB = _MSG_EPT // 128


@functools.partial(
    pl.kernel,
    out_type=jax.ShapeDtypeStruct((NCHUNK * N, CW), jnp.bfloat16),
    mesh=_sc_mesh,
    compiler_params=pltpu.CompilerParams(use_tc_tiling_on_sc=False),
    scratch_types=[
        pltpu.VMEM_SHARED((N, CW), jnp.bfloat16),
        pltpu.VMEM((_MSG_EPT,), jnp.int32),
        pltpu.VMEM((_MSG_EPT,), jnp.int32),
        pltpu.VMEM((_MSG_NB, 128), jnp.int32),
        pltpu.VMEM((128, CW), jnp.bfloat16),
        pltpu.VMEM((128, CW), jnp.bfloat16),
        pltpu.SemaphoreType.DMA,
        pltpu.SemaphoreType.DMA,
    ],
)
def _sc_scatter(src_hbm, dst_hbm, hws_hbm, out_hbm, acc,
                sidx, sadj, didx, rb0, rb1, sem0, sem1):
    c = lax.axis_index("c")
    s = lax.axis_index("s")
    pltpu.sync_copy(src_hbm.at[s], sidx)
    pltpu.sync_copy(dst_hbm.at[s], didx)

    def gat(b, rb, sem):
        return pltpu.async_copy(hws_hbm.at[sadj.at[pl.ds(b * 128, 128)]],
                                rb, sem)

    for j in range(CPS):
        off = (CPS * c + j) * N

        def adj(i, carry):
            sl = pl.ds(i * 16, 16)
            sadj[sl] = sidx[sl] + off
            return carry

        lax.fori_loop(0, _MSG_EPT // 16, adj, 0)

        pltpu.sync_copy(hws_hbm.at[pl.ds(off + s * NPG, NPG)],
                        acc.at[pl.ds(s * NPG, NPG)])
        plsc.subcore_barrier()

        gat(0, rb0, sem0)

        def wat(b, rb, sem):
            pltpu.make_async_copy(hws_hbm.at[sadj.at[pl.ds(b * 128, 128)]],
                                  rb, sem).wait()

        def body(i2, carry):
            b0 = 2 * i2
            gat(b0 + 1, rb1, sem1)
            wat(b0, rb0, sem0)
            pltpu.sync_copy(rb0, acc.at[didx.at[b0]], add=True)

            @pl.when(i2 < _MSG_NB // 2 - 1)
            def _():
                gat(b0 + 2, rb0, sem0)

            wat(b0 + 1, rb1, sem1)
            pltpu.sync_copy(rb1, acc.at[didx.at[b0 + 1]], add=True)
            return carry

        lax.fori_loop(0, _MSG_NB // 2, body, 0)
        plsc.subcore_barrier()
        pltpu.sync_copy(acc.at[pl.ds(s * NPG, NPG)],
                        out_hbm.at[pl.ds(off + s * NPG, NPG)])
        plsc.subcore_barrier()


def _run_sc_scatter(src, dst, hws4):
    src_r = src.reshape(NS, _MSG_EPT)
    dst_r = dst.reshape(NS, _MSG_NB, 128)
    return _sc_scatter(src_r, dst_r, hws4)


_K1_BLK = 256
_K1_GRID = N // _K1_BLK


def _tc_k1_body(x_ref, wemb_ref, bemb_ref, wg1_ref, dega_ref, degb_ref,
                out_ref):
    xb = x_ref[...].astype(jnp.bfloat16)
    hb = jnp.dot(xb, wemb_ref[...].astype(jnp.bfloat16),
                 preferred_element_type=jnp.float32)
    hb = (hb + bemb_ref[...]).astype(jnp.bfloat16)
    hw = jnp.dot(hb, wg1_ref[...].astype(jnp.bfloat16),
                 preferred_element_type=jnp.float32)
    deg = dega_ref[0, :, 0:1] + degb_ref[0, :, 0:1] + 1.0
    hws = (hw * lax.rsqrt(deg)).astype(jnp.bfloat16)
    for cidx in range(NCHUNK):
        out_ref[cidx] = hws[:, cidx * CW:(cidx + 1) * CW]


def _run_tc_k1(x, W_emb, b_emb, W_g1, deg2):
    da = deg2[0].reshape(_K1_GRID, _K1_BLK, 16)
    db = deg2[1].reshape(_K1_GRID, _K1_BLK, 16)
    full = lambda shape: pl.BlockSpec(shape, lambda i: (0,) * len(shape))
    out = pl.pallas_call(
        _tc_k1_body,
        grid=(_K1_GRID,),
        in_specs=[
            pl.BlockSpec((_K1_BLK, DIN), lambda i: (i, 0)),
            full((DIN, H)),
            full((1, H)),
            full((H, H)),
            pl.BlockSpec((1, _K1_BLK, 16), lambda i: (i, 0, 0)),
            pl.BlockSpec((1, _K1_BLK, 16), lambda i: (i, 0, 0)),
        ],
        out_specs=pl.BlockSpec((NCHUNK, _K1_BLK, CW), lambda i: (0, i, 0)),
        out_shape=jax.ShapeDtypeStruct((NCHUNK, N, CW), jnp.bfloat16),
    )(x, W_emb, b_emb.reshape(1, H), W_g1, da, db)
    return out.reshape(NCHUNK * N, CW)


def _tc_k2_body(op_ref, dega_ref, degb_ref, bg1_ref, aw1_ref, ab1_ref,
                aw2_ref, ab2_ref, virt_ref, ewt_ref, vw1_ref, vb1_ref,
                vw2_ref, vb2_ref, mw1_ref, mb1_ref, mw2_ref, mb2_ref,
                out_ref):
    parts = [op_ref[cidx, 0] for cidx in range(NCHUNK)]
    pre = jnp.concatenate(parts, axis=1).astype(jnp.float32)
    deg = dega_ref[0, :, 0:1] + degb_ref[0, :, 0:1] + 1.0
    gx = jax.nn.relu(pre * lax.rsqrt(deg) + bg1_ref[...])
    gxb = gx.astype(jnp.bfloat16)
    af = jnp.dot(gxb, aw1_ref[...].astype(jnp.bfloat16),
                 preferred_element_type=jnp.float32)
    af = jax.nn.relu(af + ab1_ref[...]).astype(jnp.bfloat16)
    af = jnp.dot(af, aw2_ref[...].astype(jnp.bfloat16),
                 preferred_element_type=jnp.float32)
    af = (af + ab2_ref[...]).astype(jnp.bfloat16)
    scores = lax.dot_general(af, virt_ref[0].astype(jnp.bfloat16),
                             (((1,), (1,)), ((), ())),
                             preferred_element_type=jnp.float32) * (1.0 / 16.0)
    ew = ewt_ref[0] * (1.0 + jax.nn.sigmoid(scores))
    rs = jnp.sum(ew, axis=1, keepdims=True)
    rs = jnp.where(rs == 0.0, 1.0, rs)
    ew = (ew / rs).astype(jnp.bfloat16)
    vn = lax.dot_general(ew, gxb, (((0,), (0,)), ((), ())),
                         preferred_element_type=jnp.float32)
    h1 = jnp.dot(vn.astype(jnp.bfloat16), vw1_ref[...].astype(jnp.bfloat16),
                 preferred_element_type=jnp.float32)
    h1 = jax.nn.relu(h1 + vb1_ref[...]).astype(jnp.bfloat16)
    h1 = jnp.dot(h1, vw2_ref[...].astype(jnp.bfloat16),
                 preferred_element_type=jnp.float32)
    h1 = h1 + vb2_ref[...]
    gf = jnp.mean(h1, axis=0, keepdims=True)
    o = jnp.dot(gf, mw1_ref[...], preferred_element_type=jnp.float32)
    o = jax.nn.relu(o + mb1_ref[...])
    o = jnp.dot(o, mw2_ref[...], preferred_element_type=jnp.float32)
    out_ref[0] = o + mb2_ref[...]


def _run_tc_k2(outpre4, deg2, b_g1, aW1, ab1, aW2, ab2, virt, edge_weights,
               vW1, vb1, vW2, vb2, mW1, mb1, mW2, mb2):
    op = outpre4.reshape(NCHUNK, G, NPG, CW)
    da = deg2[0].reshape(G, NPG, 16)
    db = deg2[1].reshape(G, NPG, 16)
    full = lambda shape: pl.BlockSpec(shape, lambda g: (0,) * len(shape))
    out = pl.pallas_call(
        _tc_k2_body,
        grid=(G,),
        in_specs=[
            pl.BlockSpec((NCHUNK, 1, NPG, CW), lambda g: (0, g, 0, 0)),
            pl.BlockSpec((1, NPG, 16), lambda g: (g, 0, 0)),
            pl.BlockSpec((1, NPG, 16), lambda g: (g, 0, 0)),
            full((1, H)),
            full((H, H)),
            full((1, H)),
            full((H, H)),
            full((1, H)),
            pl.BlockSpec((1, V, H), lambda g: (g, 0, 0)),
            pl.BlockSpec((1, NPG, V), lambda g: (g, 0, 0)),
            full((H, H)),
            full((1, H)),
            full((H, H)),
            full((1, H)),
            full((H, H)),
            full((1, H)),
            full((H, DOUT)),
            full((1, DOUT)),
        ],
        out_specs=pl.BlockSpec((1, 1, DOUT), lambda g: (g, 0, 0)),
        out_shape=jax.ShapeDtypeStruct((G, 1, DOUT), jnp.float32),
    )(op, da, db, b_g1.reshape(1, H), aW1, ab1.reshape(1, H), aW2,
      ab2.reshape(1, H), virt, edge_weights, vW1, vb1.reshape(1, H), vW2,
      vb2.reshape(1, H), mW1, mb1.reshape(1, H), mW2, mb2.reshape(1, DOUT))
    return out.reshape(G, DOUT)


def kernel(x, edge_index, batch, W_emb, b_emb, W_g1, b_g1, aW1, ab1, aW2,
           ab2, vW1, vb1, vW2, vb2, mW1, mb1, mW2, mb2, edge_weights):
    src = edge_index[0]
    dst = edge_index[1]
    deg2 = _run_sc_degree(dst)
    hws4 = _run_tc_k1(x, W_emb, b_emb, W_g1, deg2)
    outpre4 = _run_sc_scatter(src, dst, hws4)
    virt = jax.random.normal(jax.random.key(42), (G, V, H), dtype=jnp.float32)
    virt = virt / jnp.linalg.norm(virt, axis=2, keepdims=True)
    return _run_tc_k2(outpre4, deg2, b_g1, aW1, ab1, aW2, ab2, virt,
                      edge_weights, vW1, vb1, vW2, vb2, mW1, mb1, mW2, mb2)


# speedup vs baseline: 27.0830x; 1.0763x over previous
import functools

import jax
import jax.numpy as jnp
from jax import lax
from jax.experimental import pallas as pl
from jax.experimental.pallas import tpu as pltpu
from jax.experimental.pallas import tpu_sc as plsc

G = 16
NPG = 1024
N = G * NPG
E = 262144
DIN = 256
H = 256
DOUT = 64
V = 128

NC = 2
NS = 16
NW = NC * NS

NCHUNK = 2
CW = H // NCHUNK
CPS = NCHUNK // NC


_DEG_EPT = E // NW
_DEG_NB = _DEG_EPT // 128

_sc_mesh = plsc.VectorSubcoreMesh(core_axis_name="c", subcore_axis_name="s")


@functools.partial(
    pl.kernel,
    out_type=jax.ShapeDtypeStruct((NC, N, 16), jnp.float32),
    mesh=_sc_mesh,
    compiler_params=pltpu.CompilerParams(use_tc_tiling_on_sc=False),
    scratch_types=[
        pltpu.VMEM_SHARED((N, 16), jnp.float32),
        pltpu.VMEM((_DEG_NB, 128), jnp.int32),
        pltpu.VMEM((128, 16), jnp.float32),
    ],
)
def _sc_degree(dst_hbm, ones_hbm, zeros_hbm, deg_hbm, acc, didx, obuf):
    c = lax.axis_index("c")
    s = lax.axis_index("s")
    w = s * NC + c
    pltpu.sync_copy(zeros_hbm, acc.at[pl.ds(s * NPG, NPG)])
    pltpu.sync_copy(ones_hbm, obuf)
    pltpu.sync_copy(dst_hbm.at[w], didx)
    plsc.subcore_barrier()

    def body(j, carry):
        pltpu.sync_copy(obuf, acc.at[didx.at[j]], add=True)
        return carry

    lax.fori_loop(0, _DEG_NB, body, 0)
    plsc.subcore_barrier()
    pltpu.sync_copy(acc.at[pl.ds(s * NPG, NPG)],
                    deg_hbm.at[c, pl.ds(s * NPG, NPG)])


def _run_sc_degree(dst):
    dst_r = dst.reshape(NW, _DEG_NB, 128)
    ones = jnp.ones((128, 16), jnp.float32)
    zeros = jnp.zeros((NPG, 16), jnp.float32)
    return _sc_degree(dst_r, ones, zeros)


_MSG_EPT = E // NS
_MSG_NB = _MSG_EPT // 128


@functools.partial(
    pl.kernel,
    out_type=jax.ShapeDtypeStruct((NCHUNK, G, NPG, CW), jnp.bfloat16),
    mesh=_sc_mesh,
    compiler_params=pltpu.CompilerParams(use_tc_tiling_on_sc=False),
    scratch_types=[
        pltpu.VMEM_SHARED((N, CW), jnp.bfloat16),
        pltpu.VMEM((_MSG_EPT,), jnp.int32),
        pltpu.VMEM((_MSG_NB, 128), jnp.int32),
        pltpu.VMEM((128, CW), jnp.bfloat16),
        pltpu.VMEM((128, CW), jnp.bfloat16),
        pltpu.SemaphoreType.DMA,
        pltpu.SemaphoreType.DMA,
    ],
)
def _sc_scatter(src_hbm, dst_hbm, hws_hbm, out_hbm, acc,
                sidx, didx, rb0, rb1, sem0, sem1):
    c = lax.axis_index("c")
    s = lax.axis_index("s")
    pltpu.sync_copy(src_hbm.at[s], sidx)
    pltpu.sync_copy(dst_hbm.at[s], didx)
    chunk = hws_hbm.at[c]

    def gat(b, rb, sem):
        return pltpu.async_copy(chunk.at[sidx.at[pl.ds(b * 128, 128)]],
                                rb, sem)

    pltpu.sync_copy(chunk.at[pl.ds(s * NPG, NPG)],
                    acc.at[pl.ds(s * NPG, NPG)])
    plsc.subcore_barrier()

    gat(0, rb0, sem0)

    def wat(b, rb, sem):
        pltpu.make_async_copy(chunk.at[sidx.at[pl.ds(b * 128, 128)]],
                              rb, sem).wait()

    def body(i2, carry):
        b0 = 2 * i2
        gat(b0 + 1, rb1, sem1)
        wat(b0, rb0, sem0)
        pltpu.sync_copy(rb0, acc.at[didx.at[b0]], add=True)

        @pl.when(i2 < _MSG_NB // 2 - 1)
        def _():
            gat(b0 + 2, rb0, sem0)

        wat(b0 + 1, rb1, sem1)
        pltpu.sync_copy(rb1, acc.at[didx.at[b0 + 1]], add=True)
        return carry

    lax.fori_loop(0, _MSG_NB // 2, body, 0)
    plsc.subcore_barrier()
    pltpu.sync_copy(acc.at[pl.ds(s * NPG, NPG)], out_hbm.at[c, s])
    plsc.subcore_barrier()


def _run_sc_scatter(src, dst, hws4):
    src_r = src.reshape(NS, _MSG_EPT)
    dst_r = dst.reshape(NS, _MSG_NB, 128)
    return _sc_scatter(src_r, dst_r, hws4)


_K1_BLK = 256
_K1_GRID = N // _K1_BLK


_DEGV = N // 8
_K1_DB = _K1_BLK // 8


def _deg_rsqrt(dv):
    q = dv.shape[1]
    b = q * 8
    d = dv[0] + dv[1] + 1.0
    rown = lax.broadcasted_iota(jnp.int32, (b, q), 0)
    colq = lax.broadcasted_iota(jnp.int32, (b, q), 1)
    p = (rown // 8 == colq).astype(jnp.float32)
    r = jnp.dot(p, d, preferred_element_type=jnp.float32)
    rowb = lax.broadcasted_iota(jnp.int32, (b, 128), 0)
    lane = lax.broadcasted_iota(jnp.int32, (b, 128), 1)
    m = (lane == (rowb % 8) * 16).astype(jnp.float32)
    return lax.rsqrt(jnp.sum(r * m, axis=1, keepdims=True))


def _tc_k1_body(x_ref, wemb_ref, bemb_ref, wg1_ref, degv_ref, out_ref):
    xb = x_ref[...].astype(jnp.bfloat16)
    hb = jnp.dot(xb, wemb_ref[...].astype(jnp.bfloat16),
                 preferred_element_type=jnp.float32)
    hb = (hb + bemb_ref[...]).astype(jnp.bfloat16)
    hw = jnp.dot(hb, wg1_ref[...].astype(jnp.bfloat16),
                 preferred_element_type=jnp.float32)
    hws = (hw * _deg_rsqrt(degv_ref[...])).astype(jnp.bfloat16)
    for cidx in range(NCHUNK):
        out_ref[cidx] = hws[:, cidx * CW:(cidx + 1) * CW]


def _run_tc_k1(x, W_emb, b_emb, W_g1, deg2):
    dv = deg2.reshape(NC, _DEGV, 128)
    full = lambda shape: pl.BlockSpec(shape, lambda i: (0,) * len(shape))
    return pl.pallas_call(
        _tc_k1_body,
        grid=(_K1_GRID,),
        in_specs=[
            pl.BlockSpec((_K1_BLK, DIN), lambda i: (i, 0)),
            full((DIN, H)),
            full((1, H)),
            full((H, H)),
            pl.BlockSpec((NC, _K1_DB, 128), lambda i: (0, i, 0)),
        ],
        out_specs=pl.BlockSpec((NCHUNK, _K1_BLK, CW), lambda i: (0, i, 0)),
        out_shape=jax.ShapeDtypeStruct((NCHUNK, N, CW), jnp.bfloat16),
    )(x, W_emb, b_emb.reshape(1, H), W_g1, dv)


def _tc_k2_body(op_ref, degv_ref, bg1_ref, aw1_ref, ab1_ref,
                aw2_ref, ab2_ref, virt_ref, ewt_ref, vw1_ref, vb1_ref,
                vw2_ref, vb2_ref, mw1_ref, mb1_ref, mw2_ref, mb2_ref,
                out_ref):
    parts = [op_ref[cidx, 0] for cidx in range(NCHUNK)]
    pre = jnp.concatenate(parts, axis=1).astype(jnp.float32)
    gx = jax.nn.relu(pre * _deg_rsqrt(degv_ref[...]) + bg1_ref[...])
    gxb = gx.astype(jnp.bfloat16)
    af = jnp.dot(gxb, aw1_ref[...].astype(jnp.bfloat16),
                 preferred_element_type=jnp.float32)
    af = jax.nn.relu(af + ab1_ref[...]).astype(jnp.bfloat16)
    af = jnp.dot(af, aw2_ref[...].astype(jnp.bfloat16),
                 preferred_element_type=jnp.float32)
    af = (af + ab2_ref[...]).astype(jnp.bfloat16)
    scores = lax.dot_general(af, virt_ref[0].astype(jnp.bfloat16),
                             (((1,), (1,)), ((), ())),
                             preferred_element_type=jnp.float32) * (1.0 / 16.0)
    ew = ewt_ref[0] * (1.0 + jax.nn.sigmoid(scores))
    rs = jnp.sum(ew, axis=1, keepdims=True)
    rs = jnp.where(rs == 0.0, 1.0, rs)
    ew = (ew / rs).astype(jnp.bfloat16)
    vn = lax.dot_general(ew, gxb, (((0,), (0,)), ((), ())),
                         preferred_element_type=jnp.float32)
    h1 = jnp.dot(vn.astype(jnp.bfloat16), vw1_ref[...].astype(jnp.bfloat16),
                 preferred_element_type=jnp.float32)
    h1 = jax.nn.relu(h1 + vb1_ref[...]).astype(jnp.bfloat16)
    h1 = jnp.dot(h1, vw2_ref[...].astype(jnp.bfloat16),
                 preferred_element_type=jnp.float32)
    h1 = h1 + vb2_ref[...]
    gf = jnp.mean(h1, axis=0, keepdims=True)
    o = jnp.dot(gf, mw1_ref[...], preferred_element_type=jnp.float32)
    o = jax.nn.relu(o + mb1_ref[...])
    o = jnp.dot(o, mw2_ref[...], preferred_element_type=jnp.float32)
    out_ref[0] = o + mb2_ref[...]


_K2_DB = NPG // 8


def _run_tc_k2(op, deg2, b_g1, aW1, ab1, aW2, ab2, virt, edge_weights,
               vW1, vb1, vW2, vb2, mW1, mb1, mW2, mb2):
    dv = deg2.reshape(NC, _DEGV, 128)
    full = lambda shape: pl.BlockSpec(shape, lambda g: (0,) * len(shape))
    out = pl.pallas_call(
        _tc_k2_body,
        grid=(G,),
        in_specs=[
            pl.BlockSpec((NCHUNK, 1, NPG, CW), lambda g: (0, g, 0, 0)),
            pl.BlockSpec((NC, _K2_DB, 128), lambda g: (0, g, 0)),
            full((1, H)),
            full((H, H)),
            full((1, H)),
            full((H, H)),
            full((1, H)),
            pl.BlockSpec((1, V, H), lambda g: (g, 0, 0)),
            pl.BlockSpec((1, NPG, V), lambda g: (g, 0, 0)),
            full((H, H)),
            full((1, H)),
            full((H, H)),
            full((1, H)),
            full((H, H)),
            full((1, H)),
            full((H, DOUT)),
            full((1, DOUT)),
        ],
        out_specs=pl.BlockSpec((1, 1, DOUT), lambda g: (g, 0, 0)),
        out_shape=jax.ShapeDtypeStruct((G, 1, DOUT), jnp.float32),
    )(op, dv, b_g1.reshape(1, H), aW1, ab1.reshape(1, H), aW2,
      ab2.reshape(1, H), virt, edge_weights, vW1, vb1.reshape(1, H), vW2,
      vb2.reshape(1, H), mW1, mb1.reshape(1, H), mW2, mb2.reshape(1, DOUT))
    return out.reshape(G, DOUT)


def kernel(x, edge_index, batch, W_emb, b_emb, W_g1, b_g1, aW1, ab1, aW2,
           ab2, vW1, vb1, vW2, vb2, mW1, mb1, mW2, mb2, edge_weights):
    src = edge_index[0]
    dst = edge_index[1]
    deg2 = _run_sc_degree(dst)
    hws4 = _run_tc_k1(x, W_emb, b_emb, W_g1, deg2)
    outpre4 = _run_sc_scatter(src, dst, hws4)
    virt = jax.random.normal(jax.random.key(42), (G, V, H), dtype=jnp.float32)
    virt = virt / jnp.linalg.norm(virt, axis=2, keepdims=True)
    return _run_tc_k2(outpre4, deg2, b_g1, aW1, ab1, aW2, ab2, virt,
                      edge_weights, vW1, vb1, vW2, vb2, mW1, mb1, mW2, mb2)
